# speedup vs baseline: 1.2910x; 1.2910x over previous
import jax
import jax.numpy as jnp
from jax.experimental import pallas as pl

N = 32768
D = 1024
BLOCK = 1024


def _route_kernel(x_ref, wg_ref, bg_ref, pre_ref, post_ref):
    xb = x_ref[...]
    z = jnp.dot(xb, wg_ref[...], preferred_element_type=jnp.float32)
    z = z + bg_ref[...]
    s = jax.nn.sigmoid(z)
    m = (z > 0.0).astype(jnp.float32)
    sm = s * m
    w_pre = sm[:, 0:1] + sm[:, 1:2]
    w_post = w_pre * (m[:, 0:1] + m[:, 1:2])
    pre_ref[...] = xb * w_pre
    post_ref[...] = xb * w_post


@jax.jit
def kernel(x, Wg, bg):
    wg_p = jnp.zeros((D, 128), dtype=jnp.float32).at[:, :2].set(Wg)
    bg_p = jnp.zeros((1, 128), dtype=jnp.float32).at[0, :2].set(bg)
    grid = (N // BLOCK,)
    pre, post = pl.pallas_call(
        _route_kernel,
        grid=grid,
        in_specs=[
            pl.BlockSpec((BLOCK, D), lambda i: (i, 0)),
            pl.BlockSpec((D, 128), lambda i: (0, 0)),
            pl.BlockSpec((1, 128), lambda i: (0, 0)),
        ],
        out_specs=[
            pl.BlockSpec((BLOCK, D), lambda i: (i, 0)),
            pl.BlockSpec((BLOCK, D), lambda i: (i, 0)),
        ],
        out_shape=[
            jax.ShapeDtypeStruct((N, D), jnp.float32),
            jax.ShapeDtypeStruct((N, D), jnp.float32),
        ],
    )(x, wg_p, bg_p)
    return (pre, post)


# speedup vs baseline: 1.3233x; 1.0250x over previous
import jax
import jax.numpy as jnp
from jax.experimental import pallas as pl

N = 32768
D = 1024
BLOCK = 2048


def _route_kernel(x_ref, wg_ref, bg_ref, pre_ref, post_ref):
    xb = x_ref[...]
    z = jnp.dot(xb, wg_ref[...], preferred_element_type=jnp.float32)
    z = z + bg_ref[...]
    s = jax.nn.sigmoid(z)
    m = (z > 0.0).astype(jnp.float32)
    sm = s * m
    w_pre = sm[:, 0:1] + sm[:, 1:2]
    w_post = w_pre * (m[:, 0:1] + m[:, 1:2])
    pre_ref[...] = xb * w_pre
    post_ref[...] = xb * w_post


@jax.jit
def kernel(x, Wg, bg):
    wg_p = jnp.zeros((D, 128), dtype=jnp.float32).at[:, :2].set(Wg)
    bg_p = jnp.zeros((1, 128), dtype=jnp.float32).at[0, :2].set(bg)
    grid = (N // BLOCK,)
    pre, post = pl.pallas_call(
        _route_kernel,
        grid=grid,
        in_specs=[
            pl.BlockSpec((BLOCK, D), lambda i: (i, 0)),
            pl.BlockSpec((D, 128), lambda i: (0, 0)),
            pl.BlockSpec((1, 128), lambda i: (0, 0)),
        ],
        out_specs=[
            pl.BlockSpec((BLOCK, D), lambda i: (i, 0)),
            pl.BlockSpec((BLOCK, D), lambda i: (i, 0)),
        ],
        out_shape=[
            jax.ShapeDtypeStruct((N, D), jnp.float32),
            jax.ShapeDtypeStruct((N, D), jnp.float32),
        ],
    )(x, wg_p, bg_p)
    return (pre, post)
